# Initial kernel scaffold; baseline (speedup 1.0000x reference)
import jax
import jax.numpy as jnp
from jax.experimental import pallas as pl


def kernel(token_ids, weight):
    raise NotImplementedError("write your pallas kernel here")


# speedup vs baseline: 1.6836x; 1.6836x over previous
import functools

import jax
import jax.numpy as jnp
from jax import lax
from jax.experimental import pallas as pl
from jax.experimental.pallas import tpu as pltpu
from jax.experimental.pallas import tpu_sc as plsc

EMBED_DIM = 64
CHUNK = 128


@functools.lru_cache(maxsize=None)
def _make_kernel(n_chunks: int):
    NW = 32
    per_w = n_chunks // NW
    mesh = plsc.VectorSubcoreMesh(core_axis_name="c", subcore_axis_name="s")

    @functools.partial(
        pl.kernel,
        mesh=mesh,
        out_type=jax.ShapeDtypeStruct((n_chunks * CHUNK, EMBED_DIM), jnp.float32),
        scratch_types=[
            pltpu.VMEM((per_w, CHUNK), jnp.int32),
            pltpu.VMEM((CHUNK, EMBED_DIM), jnp.float32),
            pltpu.SemaphoreType.DMA,
        ],
        compiler_params=pltpu.CompilerParams(use_tc_tiling_on_sc=False),
    )
    def k(idx_hbm, table_hbm, out_hbm, idx_v, buf, sem):
        wid = lax.axis_index("s") * 2 + lax.axis_index("c")
        row0 = wid * per_w
        pltpu.sync_copy(idx_hbm.at[pl.ds(row0, per_w), :], idx_v)

        def body(j, carry):
            pltpu.async_copy(table_hbm.at[idx_v.at[j]], buf, sem).wait()
            pltpu.sync_copy(buf, out_hbm.at[pl.ds((row0 + j) * CHUNK, CHUNK), :])
            return carry

        lax.fori_loop(0, per_w, body, 0)

    return k


def kernel(token_ids, weight):
    b, h = token_ids.shape
    n = b * h
    idx = token_ids.reshape(n // CHUNK, CHUNK).astype(jnp.int32)
    out = _make_kernel(n // CHUNK)(idx, weight)
    return out.reshape(b, h, EMBED_DIM)


# speedup vs baseline: 1.8688x; 1.1100x over previous
import functools

import jax
import jax.numpy as jnp
from jax import lax
from jax.experimental import pallas as pl
from jax.experimental.pallas import tpu as pltpu
from jax.experimental.pallas import tpu_sc as plsc

EMBED_DIM = 64
CHUNK = 128
K = 4
NB = 2


@functools.lru_cache(maxsize=None)
def _make_kernel(n_chunks: int):
    NW = 32
    per_w = n_chunks // NW
    rounds = per_w // (K * NB)
    assert per_w % (K * NB) == 0
    mesh = plsc.VectorSubcoreMesh(core_axis_name="c", subcore_axis_name="s")

    @functools.partial(
        pl.kernel,
        mesh=mesh,
        out_type=jax.ShapeDtypeStruct((n_chunks * CHUNK, EMBED_DIM), jnp.float32),
        scratch_types=[
            pltpu.VMEM((per_w, CHUNK), jnp.int32),
            pltpu.VMEM((NB, K * CHUNK, EMBED_DIM), jnp.float32),
            pltpu.SemaphoreType.DMA,
            pltpu.SemaphoreType.DMA,
            pltpu.SemaphoreType.DMA,
            pltpu.SemaphoreType.DMA,
        ],
        compiler_params=pltpu.CompilerParams(use_tc_tiling_on_sc=False),
    )
    def k(idx_hbm, table_hbm, out_hbm, idx_v, bufs, g0, g1, o0, o1):
        gsem = (g0, g1)
        osem = (o0, o1)
        wid = lax.axis_index("s") * 2 + lax.axis_index("c")
        row0 = wid * per_w
        pltpu.sync_copy(idx_hbm.at[pl.ds(row0, per_w), :], idx_v)

        def do_group(g, b, drain_out):
            out_slice = out_hbm.at[pl.ds((row0 + g * K) * CHUNK, K * CHUNK), :]
            if drain_out:
                pltpu.make_async_copy(bufs.at[b], out_slice, osem[b]).wait()
            handles = []
            for t in range(K):
                j = g * K + t
                handles.append(
                    pltpu.async_copy(
                        table_hbm.at[idx_v.at[j]],
                        bufs.at[b, pl.ds(t * CHUNK, CHUNK), :],
                        gsem[b],
                    )
                )
            for h in handles:
                h.wait()
            pltpu.async_copy(bufs.at[b], out_slice, osem[b])

        for b in range(NB):
            do_group(b, b, drain_out=False)

        def body(r, carry):
            for b in range(NB):
                do_group(r * NB + b, b, drain_out=True)
            return carry

        lax.fori_loop(1, rounds, body, 0)

        for b in range(NB):
            g = (rounds - 1) * NB + b
            out_slice = out_hbm.at[pl.ds((row0 + g * K) * CHUNK, K * CHUNK), :]
            pltpu.make_async_copy(bufs.at[b], out_slice, osem[b]).wait()

    return k


def kernel(token_ids, weight):
    b, h = token_ids.shape
    n = b * h
    idx = token_ids.reshape(n // CHUNK, CHUNK).astype(jnp.int32)
    out = _make_kernel(n // CHUNK)(idx, weight)
    return out.reshape(b, h, EMBED_DIM)


# speedup vs baseline: 1.8759x; 1.0038x over previous
import functools

import jax
import jax.numpy as jnp
from jax import lax
from jax.experimental import pallas as pl
from jax.experimental.pallas import tpu as pltpu
from jax.experimental.pallas import tpu_sc as plsc

EMBED_DIM = 64
CHUNK = 128
NBUF = 8


@functools.lru_cache(maxsize=None)
def _make_kernel(n_chunks: int):
    NW = 32
    per_w = n_chunks // NW
    rounds = per_w // NBUF
    assert per_w % NBUF == 0 and rounds >= 2
    mesh = plsc.VectorSubcoreMesh(core_axis_name="c", subcore_axis_name="s")

    @functools.partial(
        pl.kernel,
        mesh=mesh,
        out_type=jax.ShapeDtypeStruct((n_chunks * CHUNK, EMBED_DIM), jnp.float32),
        scratch_types=[
            pltpu.VMEM((per_w, CHUNK), jnp.int32),
            pltpu.VMEM((NBUF, CHUNK, EMBED_DIM), jnp.float32),
            pltpu.SemaphoreType.DMA((NBUF,)),
            pltpu.SemaphoreType.DMA((NBUF,)),
        ],
        compiler_params=pltpu.CompilerParams(use_tc_tiling_on_sc=False),
    )
    def k(idx_hbm, table_hbm, out_hbm, idx_v, bufs, gsem, osem):
        wid = lax.axis_index("s") * 2 + lax.axis_index("c")
        row0 = wid * per_w
        pltpu.sync_copy(idx_hbm.at[pl.ds(row0, per_w), :], idx_v)

        def gather(j, b):
            return pltpu.make_async_copy(
                table_hbm.at[idx_v.at[j]], bufs.at[b], gsem.at[b]
            )

        def outcopy(j, b):
            return pltpu.make_async_copy(
                bufs.at[b],
                out_hbm.at[pl.ds((row0 + j) * CHUNK, CHUNK), :],
                osem.at[b],
            )

        for b in range(NBUF):
            gather(b, b).start()

        def step(j, b, refill):
            gather(j, b).wait()
            outcopy(j, b).start()
            if refill:
                outcopy(j, b).wait()
                gather(j + NBUF, b).start()

        def body(r, carry):
            for b in range(NBUF):
                step(r * NBUF + b, b, refill=True)
            return carry

        lax.fori_loop(0, rounds - 1, body, 0)

        for b in range(NBUF):
            step((rounds - 1) * NBUF + b, b, refill=False)
        for b in range(NBUF):
            outcopy((rounds - 1) * NBUF + b, b).wait()

    return k


def kernel(token_ids, weight):
    b, h = token_ids.shape
    n = b * h
    idx = token_ids.reshape(n // CHUNK, CHUNK).astype(jnp.int32)
    out = _make_kernel(n // CHUNK)(idx, weight)
    return out.reshape(b, h, EMBED_DIM)


# speedup vs baseline: 4.6665x; 2.4876x over previous
import functools

import jax
import jax.numpy as jnp
from jax import lax
from jax.experimental import pallas as pl
from jax.experimental.pallas import tpu as pltpu
from jax.experimental.pallas import tpu_sc as plsc

EMBED_DIM = 64
CHUNK = 128
NBUF = 8
TBLK = 16384
HALF = TBLK // 2


def _tbody(x_ref, o_ref):
    x = x_ref[...]
    xc = jnp.concatenate([x[:, 0:HALF], x[:, HALF:TBLK]], axis=0)
    o_ref[...] = xc.T


def _transpose_table(wT, n_emb):
    grid = pl.cdiv(n_emb, TBLK)
    return pl.pallas_call(
        _tbody,
        grid=(grid,),
        in_specs=[pl.BlockSpec((EMBED_DIM, TBLK), lambda i: (0, i))],
        out_specs=pl.BlockSpec((HALF, 2 * EMBED_DIM), lambda i: (i, 0)),
        out_shape=jax.ShapeDtypeStruct((grid * HALF, 2 * EMBED_DIM), jnp.float32),
    )(wT)


OT = 32


def _obody(x_ref, o_ref):
    for s in range(OT):
        xx = x_ref[pl.ds(s * 128, 128), :].T
        o_ref[0, :, 0, s, :, :] = xx[0:EMBED_DIM].reshape(8, 8, 128)
        o_ref[0, :, 1, s, :, :] = xx[EMBED_DIM:2 * EMBED_DIM].reshape(8, 8, 128)


def _untile_out(out128, n_b, n_h):
    hb = n_b // 2
    nt = hb // (128 * OT)
    return pl.pallas_call(
        _obody,
        grid=(n_h, nt),
        in_specs=[pl.BlockSpec((128 * OT, 128), lambda h, t: (h * nt + t, 0))],
        out_specs=pl.BlockSpec(
            (1, 8, 2, OT, 8, 128), lambda h, t: (h, 0, 0, t, 0, 0)
        ),
        out_shape=jax.ShapeDtypeStruct(
            (n_h, 8, 2, hb // 128, 8, 128), jnp.float32
        ),
    )(out128)


@functools.lru_cache(maxsize=None)
def _make_kernel(n_chunks: int, n_emb: int):
    NW = 32
    per_w = n_chunks // NW
    rounds = per_w // NBUF
    assert per_w % NBUF == 0 and rounds >= 2
    mesh = plsc.VectorSubcoreMesh(core_axis_name="c", subcore_axis_name="s")

    @functools.partial(
        pl.kernel,
        mesh=mesh,
        out_type=jax.ShapeDtypeStruct((n_chunks * CHUNK, EMBED_DIM), jnp.float32),
        scratch_types=[
            pltpu.VMEM((per_w, CHUNK), jnp.int32),
            pltpu.VMEM((NBUF, CHUNK, EMBED_DIM), jnp.float32),
            pltpu.SemaphoreType.DMA((NBUF,)),
            pltpu.SemaphoreType.DMA((NBUF,)),
        ],
        compiler_params=pltpu.CompilerParams(use_tc_tiling_on_sc=False),
    )
    def k(idx_hbm, table_hbm, out_hbm, idx_v, bufs, gsem, osem):
        wid = lax.axis_index("s") * 2 + lax.axis_index("c")
        row0 = wid * per_w
        pltpu.sync_copy(idx_hbm.at[pl.ds(row0, per_w), :], idx_v)

        def gather(j, b):
            return pltpu.make_async_copy(
                table_hbm.at[idx_v.at[j]], bufs.at[b], gsem.at[b]
            )

        def outcopy(j, b):
            return pltpu.make_async_copy(
                bufs.at[b],
                out_hbm.at[pl.ds((row0 + j) * CHUNK, CHUNK), :],
                osem.at[b],
            )

        for b in range(NBUF):
            gather(b, b).start()

        def step(j, b, refill):
            gather(j, b).wait()
            outcopy(j, b).start()
            if refill:
                outcopy(j, b).wait()
                gather(j + NBUF, b).start()

        def body(r, carry):
            for b in range(NBUF):
                step(r * NBUF + b, b, refill=True)
            return carry

        lax.fori_loop(0, rounds - 1, body, 0)

        for b in range(NBUF):
            step((rounds - 1) * NBUF + b, b, refill=False)
        for b in range(NBUF):
            outcopy((rounds - 1) * NBUF + b, b).wait()

    return k


def kernel(token_ids, weight):
    b, h = token_ids.shape
    n = b * h
    n_emb = weight.shape[0]
    n_pad = pl.cdiv(n_emb, TBLK) * TBLK
    w2 = _transpose_table(weight.T, n_emb)
    w = w2.reshape(n_pad, EMBED_DIM)
    g = (
        token_ids.T.reshape(h, 2, 128, b // 256)
        .transpose(0, 2, 3, 1)
        .reshape(-1)
    )
    g = g.astype(jnp.int32)
    k = g & (TBLK - 1)
    rho = g - k + 2 * (k & (HALF - 1)) + (k // HALF)
    idx = rho.reshape(n // CHUNK, CHUNK)
    out = _make_kernel(n // CHUNK, n_emb)(idx, w)
    out5 = _untile_out(out.reshape(n // 2, 128), b, h)
    return (
        out5.reshape(h, 8, b // 128, 8, 128)
        .transpose(2, 4, 0, 1, 3)
        .reshape(b, h, EMBED_DIM)
    )


# speedup vs baseline: 4.8581x; 1.0410x over previous
import functools

import jax
import jax.numpy as jnp
from jax import lax
from jax.experimental import pallas as pl
from jax.experimental.pallas import tpu as pltpu
from jax.experimental.pallas import tpu_sc as plsc

EMBED_DIM = 64
CHUNK = 128
NBUF = 10
TBLK = 16384
HALF = TBLK // 2


def _tbody(x_ref, o_ref):
    x = x_ref[...]
    xc = jnp.concatenate([x[:, 0:HALF], x[:, HALF:TBLK]], axis=0)
    o_ref[...] = xc.T


def _transpose_table(wT, n_emb):
    grid = pl.cdiv(n_emb, TBLK)
    return pl.pallas_call(
        _tbody,
        grid=(grid,),
        in_specs=[pl.BlockSpec((EMBED_DIM, TBLK), lambda i: (0, i))],
        out_specs=pl.BlockSpec((HALF, 2 * EMBED_DIM), lambda i: (i, 0)),
        out_shape=jax.ShapeDtypeStruct((grid * HALF, 2 * EMBED_DIM), jnp.float32),
    )(wT)


OT = 32


def _obody(x_ref, o_ref):
    for s in range(OT):
        xx = x_ref[pl.ds(s * 128, 128), :].T
        o_ref[0, :, 0, s, :, :] = xx[0:EMBED_DIM].reshape(8, 8, 128)
        o_ref[0, :, 1, s, :, :] = xx[EMBED_DIM:2 * EMBED_DIM].reshape(8, 8, 128)


def _obody_alias(x_ref, _, o_ref):
    _obody(x_ref, o_ref)


def _untile_out(out128, n_b, n_h, all_h, h_off, full=None):
    hb = n_b // 2
    nt = hb // (128 * OT)
    out_shape = jax.ShapeDtypeStruct((all_h, 8, 2, hb // 128, 8, 128), jnp.float32)
    in_specs = [pl.BlockSpec((128 * OT, 128), lambda h, t: (h * nt + t, 0))]
    args = (out128,)
    body = _obody
    aliases = {}
    if full is not None:
        in_specs.append(pl.BlockSpec(memory_space=pl.ANY))
        args = (out128, full)
        body = _obody_alias
        aliases = {1: 0}
    return pl.pallas_call(
        body,
        grid=(n_h, nt),
        in_specs=in_specs,
        out_specs=pl.BlockSpec(
            (1, 8, 2, OT, 8, 128), lambda h, t: (h + h_off, 0, 0, t, 0, 0)
        ),
        out_shape=out_shape,
        input_output_aliases=aliases,
    )(*args)


@functools.lru_cache(maxsize=None)
def _make_kernel(n_chunks: int, n_emb: int):
    NW = 32
    per_w = n_chunks // NW
    rounds = per_w // NBUF
    assert per_w % NBUF == 0 and rounds >= 2
    mesh = plsc.VectorSubcoreMesh(core_axis_name="c", subcore_axis_name="s")

    @functools.partial(
        pl.kernel,
        mesh=mesh,
        out_type=jax.ShapeDtypeStruct((n_chunks * CHUNK, EMBED_DIM), jnp.float32),
        scratch_types=[
            pltpu.VMEM((per_w, CHUNK), jnp.int32),
            pltpu.VMEM((NBUF, CHUNK, EMBED_DIM), jnp.float32),
            pltpu.SemaphoreType.DMA((NBUF,)),
            pltpu.SemaphoreType.DMA((NBUF,)),
        ],
        compiler_params=pltpu.CompilerParams(use_tc_tiling_on_sc=False),
    )
    def k(idx_hbm, table_hbm, out_hbm, idx_v, bufs, gsem, osem):
        wid = lax.axis_index("s") * 2 + lax.axis_index("c")
        row0 = wid * per_w
        pltpu.sync_copy(idx_hbm.at[pl.ds(row0, per_w), :], idx_v)

        def gather(j, b):
            return pltpu.make_async_copy(
                table_hbm.at[idx_v.at[j]], bufs.at[b], gsem.at[b]
            )

        def outcopy(j, b):
            return pltpu.make_async_copy(
                bufs.at[b],
                out_hbm.at[pl.ds((row0 + j) * CHUNK, CHUNK), :],
                osem.at[b],
            )

        for b in range(NBUF):
            gather(b, b).start()

        def step(j, b, refill):
            gather(j, b).wait()
            outcopy(j, b).start()
            if refill:
                outcopy(j, b).wait()
                gather(j + NBUF, b).start()

        def body(r, carry):
            for b in range(NBUF):
                step(r * NBUF + b, b, refill=True)
            return carry

        lax.fori_loop(0, rounds - 1, body, 0)

        for b in range(NBUF):
            step((rounds - 1) * NBUF + b, b, refill=False)
        for b in range(NBUF):
            outcopy((rounds - 1) * NBUF + b, b).wait()

    return k


def kernel(token_ids, weight):
    b, h = token_ids.shape
    n = b * h
    n_emb = weight.shape[0]
    n_pad = pl.cdiv(n_emb, TBLK) * TBLK
    w2 = _transpose_table(weight.T, n_emb)
    w = w2.reshape(n_pad, EMBED_DIM)
    hh = h // 2
    tT = token_ids.T.astype(jnp.int32)
    full = None
    for half_i in range(2):
        g = (
            tT[half_i * hh:(half_i + 1) * hh]
            .reshape(hh, 2, 128, b // 256)
            .transpose(0, 2, 3, 1)
            .reshape(-1)
        )
        k = g & (TBLK - 1)
        rho = g - k + 2 * (k & (HALF - 1)) + (k // HALF)
        idx = rho.reshape(-1, CHUNK)
        out = _make_kernel(idx.shape[0], n_emb)(idx, w)
        full = _untile_out(
            out.reshape(-1, 128), b, hh, h, half_i * hh, full=full
        )
    return (
        full.reshape(h, 8, b // 128, 8, 128)
        .transpose(2, 4, 0, 1, 3)
        .reshape(b, h, EMBED_DIM)
    )


# speedup vs baseline: 4.8959x; 1.0078x over previous
import functools

import jax
import jax.numpy as jnp
from jax import lax
from jax.experimental import pallas as pl
from jax.experimental.pallas import tpu as pltpu
from jax.experimental.pallas import tpu_sc as plsc

EMBED_DIM = 64
CHUNK = 128
NBUF = 10
TBLK = 16384
HALF = TBLK // 2


def _tbody(x_ref, o_ref):
    x = x_ref[...]
    xc = jnp.concatenate([x[:, 0:HALF], x[:, HALF:TBLK]], axis=0)
    o_ref[...] = xc.T


def _transpose_table(wT, n_emb):
    grid = pl.cdiv(n_emb, TBLK)
    return pl.pallas_call(
        _tbody,
        grid=(grid,),
        in_specs=[pl.BlockSpec((EMBED_DIM, TBLK), lambda i: (0, i))],
        out_specs=pl.BlockSpec((HALF, 2 * EMBED_DIM), lambda i: (i, 0)),
        out_shape=jax.ShapeDtypeStruct((grid * HALF, 2 * EMBED_DIM), jnp.float32),
    )(wT)


OT = 32


def _obody(x_ref, o_ref):
    for s in range(OT):
        xx = x_ref[pl.ds(s * 128, 128), :].T
        o_ref[0, :, 0, s, :, :] = xx[0:EMBED_DIM].reshape(8, 8, 128)
        o_ref[0, :, 1, s, :, :] = xx[EMBED_DIM:2 * EMBED_DIM].reshape(8, 8, 128)


def _obody_alias(x_ref, _, o_ref):
    _obody(x_ref, o_ref)


def _untile_out(out128, n_b, n_h, all_h, h_off, full=None):
    hb = n_b // 2
    nt = hb // (128 * OT)
    out_shape = jax.ShapeDtypeStruct((all_h, 8, 2, hb // 128, 8, 128), jnp.float32)
    in_specs = [pl.BlockSpec((128 * OT, 128), lambda h, t: (h * nt + t, 0))]
    args = (out128,)
    body = _obody
    aliases = {}
    if full is not None:
        in_specs.append(pl.BlockSpec(memory_space=pl.ANY))
        args = (out128, full)
        body = _obody_alias
        aliases = {1: 0}
    return pl.pallas_call(
        body,
        grid=(n_h, nt),
        in_specs=in_specs,
        out_specs=pl.BlockSpec(
            (1, 8, 2, OT, 8, 128), lambda h, t: (h + h_off, 0, 0, t, 0, 0)
        ),
        out_shape=out_shape,
        input_output_aliases=aliases,
    )(*args)


@functools.lru_cache(maxsize=None)
def _make_kernel(n_chunks: int, n_emb: int):
    NW = 32
    per_w = n_chunks // NW
    rounds = per_w // NBUF
    assert per_w % NBUF == 0 and rounds >= 2
    mesh = plsc.VectorSubcoreMesh(core_axis_name="c", subcore_axis_name="s")

    @functools.partial(
        pl.kernel,
        mesh=mesh,
        out_type=jax.ShapeDtypeStruct((n_chunks * CHUNK, EMBED_DIM), jnp.float32),
        scratch_types=[
            pltpu.VMEM((per_w, CHUNK), jnp.int32),
            pltpu.VMEM((NBUF, CHUNK, EMBED_DIM), jnp.float32),
            pltpu.SemaphoreType.DMA((NBUF,)),
            pltpu.SemaphoreType.DMA((NBUF,)),
        ],
        compiler_params=pltpu.CompilerParams(use_tc_tiling_on_sc=False),
    )
    def k(idx_hbm, table_hbm, out_hbm, idx_v, bufs, gsem, osem):
        wid = lax.axis_index("s") * 2 + lax.axis_index("c")
        row0 = wid * per_w
        pltpu.sync_copy(idx_hbm.at[pl.ds(row0, per_w), :], idx_v)

        def gather(j, b):
            return pltpu.make_async_copy(
                table_hbm.at[idx_v.at[j]], bufs.at[b], gsem.at[b]
            )

        def outcopy(j, b):
            return pltpu.make_async_copy(
                bufs.at[b],
                out_hbm.at[pl.ds((row0 + j) * CHUNK, CHUNK), :],
                osem.at[b],
            )

        for b in range(NBUF):
            gather(b, b).start()

        def step(j, b, refill):
            gather(j, b).wait()
            outcopy(j, b).start()
            if refill:
                outcopy(j, b).wait()
                gather(j + NBUF, b).start()

        def body(r, carry):
            for b in range(NBUF):
                step(r * NBUF + b, b, refill=True)
            return carry

        lax.fori_loop(0, rounds - 1, body, 0)

        for b in range(NBUF):
            step((rounds - 1) * NBUF + b, b, refill=False)
        for b in range(NBUF):
            outcopy((rounds - 1) * NBUF + b, b).wait()

    return k


def kernel(token_ids, weight):
    b, h = token_ids.shape
    n = b * h
    n_emb = weight.shape[0]
    n_pad = pl.cdiv(n_emb, TBLK) * TBLK
    w2 = _transpose_table(weight.T, n_emb)
    w = w2.reshape(n_pad, EMBED_DIM)
    hh = h // 5
    tT = token_ids.T.astype(jnp.int32)
    full = None
    for half_i in range(5):
        g = (
            tT[half_i * hh:(half_i + 1) * hh]
            .reshape(hh, 2, 128, b // 256)
            .transpose(0, 2, 3, 1)
            .reshape(-1)
        )
        k = g & (TBLK - 1)
        rho = g - k + 2 * (k & (HALF - 1)) + (k // HALF)
        idx = rho.reshape(-1, CHUNK)
        out = _make_kernel(idx.shape[0], n_emb)(idx, w)
        full = _untile_out(
            out.reshape(-1, 128), b, hh, h, half_i * hh, full=full
        )
    return (
        full.reshape(h, 8, b // 128, 8, 128)
        .transpose(2, 4, 0, 1, 3)
        .reshape(b, h, EMBED_DIM)
    )
